# baseline (speedup 1.0000x reference)
import functools

import jax
import jax.numpy as jnp
from jax import lax
from jax.experimental import pallas as pl
from jax.experimental.pallas import tpu as pltpu
from jax.experimental.pallas import tpu_sc as plsc

V = 1000
D = 1000
DP = 1024
NW = 32
CH = 32
RB = 400
T_LEN = 50


def _lse_body(tab_ref, out_ref):
    x = tab_ref[...]
    m = jnp.max(x, axis=1, keepdims=True)
    s = jnp.sum(jnp.exp(x - m), axis=1, keepdims=True)
    out_ref[...] = m + jnp.log(s)


def _sc_body(table_p, fidx, out,
             idx_v, rows0_v, rows1_v,
             gs0, gs1, ws0, ws1):
    wid = lax.axis_index("s") * 2 + lax.axis_index("c")
    nrows = out.shape[0] // NW
    nch = nrows // CH
    base = wid * nrows
    pltpu.sync_copy(fidx.at[pl.ds(base, nrows)], idx_v)

    bufs = (rows0_v, rows1_v)
    gsems = (gs0, gs1)
    wsems = (ws0, ws1)

    def start_gather(c, b):
        isl = idx_v.at[pl.ds(pl.multiple_of(c * CH, 8), CH)]
        pltpu.async_copy(table_p.at[isl], bufs[b], gsems[b])

    def start_write(c, b):
        pltpu.async_copy(bufs[b], out.at[pl.ds(base + c * CH, CH)],
                         wsems[b])

    def wait_write(c, b):
        pltpu.make_async_copy(bufs[b], out.at[pl.ds(base, CH)],
                              wsems[b]).wait()

    def wait_gather(b):
        pltpu.make_async_copy(table_p.at[pl.ds(0, CH)], bufs[b],
                              gsems[b]).wait()

    start_gather(0, 0)

    def pair(m, carry):
        c0 = 2 * m
        c1 = c0 + 1
        @pl.when(m > 0)
        def _():
            wait_write(c0 - 1, 1)
        start_gather(c1, 1)
        wait_gather(0)
        start_write(c0, 0)
        wait_write(c0, 0)

        @pl.when(c1 + 1 < nch)
        def _():
            start_gather(c1 + 1, 0)
        wait_gather(1)
        start_write(c1, 1)
        return carry

    lax.fori_loop(0, nch // 2, pair, jnp.int32(0))
    wait_write(nch - 1, 1)


def _finish_body(in_ref, tgt_ref, out_ref, part_ref):
    x = in_ref[:, pl.ds(0, D)]
    out_ref[...] = x.reshape(RB // T_LEN, T_LEN, D)
    m = jnp.max(x, axis=1, keepdims=True)
    s = jnp.sum(jnp.exp(x - m), axis=1, keepdims=True)
    lse = m + jnp.log(s)
    cols = lax.broadcasted_iota(jnp.int32, (RB, D), 1)
    tl = jnp.sum(jnp.where(cols == tgt_ref[...], x, 0.0), axis=1,
                 keepdims=True)
    part_ref[...] = jnp.reshape(jnp.sum(lse - tl), (1, 1, 1))


def kernel(input_idx, target, embedding_table):
    B, T = input_idx.shape
    N = B * T
    fidx = input_idx.reshape(N)
    table_p = jnp.pad(embedding_table, ((0, 0), (0, DP - D)))

    mesh = plsc.VectorSubcoreMesh(core_axis_name="c", subcore_axis_name="s")
    sc = pl.kernel(
        _sc_body,
        out_type=jax.ShapeDtypeStruct((N, DP), jnp.float32),
        mesh=mesh,
        scratch_types=[
            pltpu.VMEM((N // NW,), jnp.int32),
            pltpu.VMEM((CH, DP), jnp.float32),
            pltpu.VMEM((CH, DP), jnp.float32),
            pltpu.SemaphoreType.DMA,
            pltpu.SemaphoreType.DMA,
            pltpu.SemaphoreType.DMA,
            pltpu.SemaphoreType.DMA,
        ],
    )
    padded = sc(table_p, fidx)

    nb = N // RB
    logits, part = pl.pallas_call(
        _finish_body,
        grid=(nb,),
        in_specs=[
            pl.BlockSpec((RB, DP), lambda i: (i, 0)),
            pl.BlockSpec((RB, 1), lambda i: (i, 0)),
        ],
        out_specs=[
            pl.BlockSpec((RB // T_LEN, T_LEN, D), lambda i: (i, 0, 0)),
            pl.BlockSpec((1, 1, 1), lambda i: (i, 0, 0)),
        ],
        out_shape=[
            jax.ShapeDtypeStruct((B, T, D), jnp.float32),
            jax.ShapeDtypeStruct((nb, 1, 1), jnp.float32),
        ],
    )(padded, target.reshape(N, 1))
    loss = jnp.sum(part) / jnp.float32(N)
    return logits, loss


# speedup vs baseline: 1.0399x; 1.0399x over previous
import functools

import jax
import jax.numpy as jnp
from jax import lax
from jax.experimental import pallas as pl
from jax.experimental.pallas import tpu as pltpu
from jax.experimental.pallas import tpu_sc as plsc

V = 1000
D = 1000
DP = 1024
NW = 32
CH = 32
RB = 400
T_LEN = 50


def _lse_body(tab_ref, out_ref):
    x = tab_ref[...]
    m = jnp.max(x, axis=1, keepdims=True)
    s = jnp.sum(jnp.exp(x - m), axis=1, keepdims=True)
    out_ref[...] = m + jnp.log(s)


def _sc_body(table_p, lse_p, fidx, ftgt, out, partial,
             idx_v, tgt_v, lse_v, rows0_v, rows1_v, acc_v,
             gs0, gs1, ws0, ws1):
    wid = lax.axis_index("s") * 2 + lax.axis_index("c")
    nrows = out.shape[0] // NW
    nch = nrows // CH
    base = wid * nrows
    iota = lax.iota(jnp.int32, 16)
    pltpu.sync_copy(lse_p, lse_v)
    pltpu.sync_copy(fidx.at[pl.ds(base, nrows)], idx_v)
    pltpu.sync_copy(ftgt.at[pl.ds(base, nrows)], tgt_v)

    bufs = (rows0_v, rows1_v)
    gsems = (gs0, gs1)
    wsems = (ws0, ws1)

    def start_gather(c, b):
        isl = idx_v.at[pl.ds(pl.multiple_of(c * CH, 8), CH)]
        pltpu.async_copy(table_p.at[isl], bufs[b], gsems[b])

    def start_write(c, b):
        pltpu.async_copy(bufs[b], out.at[pl.ds(base + c * CH, CH)],
                         wsems[b])

    def wait_write(c, b):
        pltpu.make_async_copy(bufs[b], out.at[pl.ds(base, CH)],
                              wsems[b]).wait()

    def wait_gather(b):
        pltpu.make_async_copy(table_p.at[pl.ds(0, CH)], bufs[b],
                              gsems[b]).wait()

    def loss_math(c, b, acc):
        zero = jnp.zeros((16,), jnp.float32)
        regs = []
        for j in range(CH // 16):
            o = pl.multiple_of(c * CH + 16 * j, 16)
            regs.append((idx_v[pl.ds(o, 16)], tgt_v[pl.ds(o, 16)]))
        for r in range(CH):
            j, k = divmod(r, 16)
            ii = regs[j][0][k]
            tg = regs[j][1][k]
            lvec = lse_v[pl.ds(pl.multiple_of((ii >> 4) << 4, 16), 16)]
            tvec = bufs[b][r, pl.ds(pl.multiple_of((tg >> 4) << 4, 16), 16)]
            acc = (acc
                   + jnp.where(iota == (ii & 15), lvec, zero)
                   - jnp.where(iota == (tg & 15), tvec, zero))
        return acc

    start_gather(0, 0)

    def pair(m, acc):
        c0 = 2 * m
        c1 = c0 + 1
        @pl.when(m > 0)
        def _():
            wait_write(c0 - 1, 1)
        start_gather(c1, 1)
        wait_gather(0)
        start_write(c0, 0)
        acc = loss_math(c0, 0, acc)
        wait_write(c0, 0)

        @pl.when(c1 + 1 < nch)
        def _():
            start_gather(c1 + 1, 0)
        wait_gather(1)
        start_write(c1, 1)
        return loss_math(c1, 1, acc)

    acc = lax.fori_loop(0, nch // 2, pair, jnp.zeros((16,), jnp.float32))
    wait_write(nch - 1, 1)
    acc_v[...] = acc
    pltpu.sync_copy(acc_v, partial.at[wid])


def _finish_body(in_ref, out_ref):
    out_ref[...] = in_ref[:, pl.ds(0, D)].reshape(RB // T_LEN, T_LEN, D)


def kernel(input_idx, target, embedding_table):
    B, T = input_idx.shape
    N = B * T
    fidx = input_idx.reshape(N)
    ftgt = target.reshape(N)
    table_p = jnp.pad(embedding_table, ((0, 0), (0, DP - D)))
    lse = pl.pallas_call(
        _lse_body,
        out_shape=jax.ShapeDtypeStruct((V, 1), jnp.float32),
    )(embedding_table).reshape(V)
    lse_p = jnp.pad(lse, (0, DP - V))

    mesh = plsc.VectorSubcoreMesh(core_axis_name="c", subcore_axis_name="s")
    sc = pl.kernel(
        _sc_body,
        out_type=[jax.ShapeDtypeStruct((N, DP), jnp.float32),
                  jax.ShapeDtypeStruct((NW, 16), jnp.float32)],
        mesh=mesh,
        scratch_types=[
            pltpu.VMEM((N // NW,), jnp.int32),
            pltpu.VMEM((N // NW,), jnp.int32),
            pltpu.VMEM((DP,), jnp.float32),
            pltpu.VMEM((CH, DP), jnp.float32),
            pltpu.VMEM((CH, DP), jnp.float32),
            pltpu.VMEM((16,), jnp.float32),
            pltpu.SemaphoreType.DMA,
            pltpu.SemaphoreType.DMA,
            pltpu.SemaphoreType.DMA,
            pltpu.SemaphoreType.DMA,
        ],
    )
    padded, partial = sc(table_p, lse_p, fidx, ftgt)

    nb = N // RB
    logits = pl.pallas_call(
        _finish_body,
        grid=(nb,),
        in_specs=[pl.BlockSpec((RB, DP), lambda i: (i, 0))],
        out_specs=pl.BlockSpec((RB // T_LEN, T_LEN, D), lambda i: (i, 0, 0)),
        out_shape=jax.ShapeDtypeStruct((B, T, D), jnp.float32),
    )(padded)
    loss = jnp.sum(partial) / jnp.float32(N)
    return logits, loss


# speedup vs baseline: 1.0831x; 1.0416x over previous
import functools

import jax
import jax.numpy as jnp
from jax import lax
from jax.experimental import pallas as pl
from jax.experimental.pallas import tpu as pltpu
from jax.experimental.pallas import tpu_sc as plsc

V = 1000
D = 1000
DP = 1024
NW = 32
CH = 32
RB = 800
T_LEN = 50


def _lse_body(tab_ref, out_ref):
    x = tab_ref[...]
    m = jnp.max(x, axis=1, keepdims=True)
    s = jnp.sum(jnp.exp(x - m), axis=1, keepdims=True)
    out_ref[...] = m + jnp.log(s)


def _sc_body(table_p, lse_p, fidx, ftgt, out, partial,
             idx_v, tgt_v, lse_v, rows0_v, rows1_v, acc_v,
             gs0, gs1, ws0, ws1):
    wid = lax.axis_index("s") * 2 + lax.axis_index("c")
    nrows = out.shape[0] // NW
    nch = nrows // CH
    base = wid * nrows
    iota = lax.iota(jnp.int32, 16)
    pltpu.sync_copy(lse_p, lse_v)
    pltpu.sync_copy(fidx.at[pl.ds(base, nrows)], idx_v)
    pltpu.sync_copy(ftgt.at[pl.ds(base, nrows)], tgt_v)

    bufs = (rows0_v, rows1_v)
    gsems = (gs0, gs1)
    wsems = (ws0, ws1)

    def start_gather(c, b):
        isl = idx_v.at[pl.ds(pl.multiple_of(c * CH, 8), CH)]
        pltpu.async_copy(table_p.at[isl], bufs[b], gsems[b])

    def start_write(c, b):
        pltpu.async_copy(bufs[b], out.at[pl.ds(base + c * CH, CH)],
                         wsems[b])

    def wait_write(c, b):
        pltpu.make_async_copy(bufs[b], out.at[pl.ds(base, CH)],
                              wsems[b]).wait()

    def wait_gather(b):
        pltpu.make_async_copy(table_p.at[pl.ds(0, CH)], bufs[b],
                              gsems[b]).wait()

    def loss_math(c, b, acc):
        zero = jnp.zeros((16,), jnp.float32)
        regs = []
        for j in range(CH // 16):
            o = pl.multiple_of(c * CH + 16 * j, 16)
            regs.append((idx_v[pl.ds(o, 16)], tgt_v[pl.ds(o, 16)]))
        for r in range(CH):
            j, k = divmod(r, 16)
            ii = regs[j][0][k]
            tg = regs[j][1][k]
            lvec = lse_v[pl.ds(pl.multiple_of((ii >> 4) << 4, 16), 16)]
            tvec = bufs[b][r, pl.ds(pl.multiple_of((tg >> 4) << 4, 16), 16)]
            acc = (acc
                   + jnp.where(iota == (ii & 15), lvec, zero)
                   - jnp.where(iota == (tg & 15), tvec, zero))
        return acc

    start_gather(0, 0)

    def pair(m, acc):
        c0 = 2 * m
        c1 = c0 + 1
        @pl.when(m > 0)
        def _():
            wait_write(c0 - 1, 1)
        start_gather(c1, 1)
        wait_gather(0)
        start_write(c0, 0)
        acc = loss_math(c0, 0, acc)
        wait_write(c0, 0)

        @pl.when(c1 + 1 < nch)
        def _():
            start_gather(c1 + 1, 0)
        wait_gather(1)
        start_write(c1, 1)
        return loss_math(c1, 1, acc)

    acc = lax.fori_loop(0, nch // 2, pair, jnp.zeros((16,), jnp.float32))
    wait_write(nch - 1, 1)
    acc_v[...] = acc
    pltpu.sync_copy(acc_v, partial.at[wid])


def _finish_body(in_ref, out_ref):
    out_ref[...] = in_ref[:, pl.ds(0, D)].reshape(RB // T_LEN, T_LEN, D)


def kernel(input_idx, target, embedding_table):
    B, T = input_idx.shape
    N = B * T
    fidx = input_idx.reshape(N)
    ftgt = target.reshape(N)
    table_p = jnp.pad(embedding_table, ((0, 0), (0, DP - D)))
    lse = pl.pallas_call(
        _lse_body,
        out_shape=jax.ShapeDtypeStruct((V, 1), jnp.float32),
    )(embedding_table).reshape(V)
    lse_p = jnp.pad(lse, (0, DP - V))

    mesh = plsc.VectorSubcoreMesh(core_axis_name="c", subcore_axis_name="s")
    sc = pl.kernel(
        _sc_body,
        out_type=[jax.ShapeDtypeStruct((N, DP), jnp.float32),
                  jax.ShapeDtypeStruct((NW, 16), jnp.float32)],
        mesh=mesh,
        scratch_types=[
            pltpu.VMEM((N // NW,), jnp.int32),
            pltpu.VMEM((N // NW,), jnp.int32),
            pltpu.VMEM((DP,), jnp.float32),
            pltpu.VMEM((CH, DP), jnp.float32),
            pltpu.VMEM((CH, DP), jnp.float32),
            pltpu.VMEM((16,), jnp.float32),
            pltpu.SemaphoreType.DMA,
            pltpu.SemaphoreType.DMA,
            pltpu.SemaphoreType.DMA,
            pltpu.SemaphoreType.DMA,
        ],
    )
    padded, partial = sc(table_p, lse_p, fidx, ftgt)

    nb = N // RB
    logits = pl.pallas_call(
        _finish_body,
        grid=(nb,),
        in_specs=[pl.BlockSpec((RB, DP), lambda i: (i, 0))],
        out_specs=pl.BlockSpec((RB // T_LEN, T_LEN, D), lambda i: (i, 0, 0)),
        out_shape=jax.ShapeDtypeStruct((B, T, D), jnp.float32),
    )(padded)
    loss = jnp.sum(partial) / jnp.float32(N)
    return logits, loss


# speedup vs baseline: 1.0885x; 1.0050x over previous
import functools

import jax
import jax.numpy as jnp
from jax import lax
from jax.experimental import pallas as pl
from jax.experimental.pallas import tpu as pltpu
from jax.experimental.pallas import tpu_sc as plsc

V = 1000
D = 1000
DP = 1024
NW = 32
CH = 32
RB = 1600
T_LEN = 50


def _lse_body(tab_ref, out_ref):
    x = tab_ref[...]
    m = jnp.max(x, axis=1, keepdims=True)
    s = jnp.sum(jnp.exp(x - m), axis=1, keepdims=True)
    out_ref[...] = m + jnp.log(s)


def _sc_body(table_p, lse_p, fidx, ftgt, out, partial,
             idx_v, tgt_v, lse_v, rows0_v, rows1_v, acc_v,
             gs0, gs1, ws0, ws1):
    wid = lax.axis_index("s") * 2 + lax.axis_index("c")
    nrows = out.shape[0] // NW
    nch = nrows // CH
    base = wid * nrows
    iota = lax.iota(jnp.int32, 16)
    pltpu.sync_copy(lse_p, lse_v)
    pltpu.sync_copy(fidx.at[pl.ds(base, nrows)], idx_v)
    pltpu.sync_copy(ftgt.at[pl.ds(base, nrows)], tgt_v)

    bufs = (rows0_v, rows1_v)
    gsems = (gs0, gs1)
    wsems = (ws0, ws1)

    def start_gather(c, b):
        isl = idx_v.at[pl.ds(pl.multiple_of(c * CH, 8), CH)]
        pltpu.async_copy(table_p.at[isl], bufs[b], gsems[b])

    def start_write(c, b):
        pltpu.async_copy(bufs[b], out.at[pl.ds(base + c * CH, CH)],
                         wsems[b])

    def wait_write(c, b):
        pltpu.make_async_copy(bufs[b], out.at[pl.ds(base, CH)],
                              wsems[b]).wait()

    def wait_gather(b):
        pltpu.make_async_copy(table_p.at[pl.ds(0, CH)], bufs[b],
                              gsems[b]).wait()

    def loss_math(c, b, acc):
        zero = jnp.zeros((16,), jnp.float32)
        regs = []
        for j in range(CH // 16):
            o = pl.multiple_of(c * CH + 16 * j, 16)
            regs.append((idx_v[pl.ds(o, 16)], tgt_v[pl.ds(o, 16)]))
        for r in range(CH):
            j, k = divmod(r, 16)
            ii = regs[j][0][k]
            tg = regs[j][1][k]
            lvec = lse_v[pl.ds(pl.multiple_of((ii >> 4) << 4, 16), 16)]
            tvec = bufs[b][r, pl.ds(pl.multiple_of((tg >> 4) << 4, 16), 16)]
            acc = (acc
                   + jnp.where(iota == (ii & 15), lvec, zero)
                   - jnp.where(iota == (tg & 15), tvec, zero))
        return acc

    start_gather(0, 0)

    def pair(m, acc):
        c0 = 2 * m
        c1 = c0 + 1
        @pl.when(m > 0)
        def _():
            wait_write(c0 - 1, 1)
        start_gather(c1, 1)
        wait_gather(0)
        start_write(c0, 0)
        acc = loss_math(c0, 0, acc)
        wait_write(c0, 0)

        @pl.when(c1 + 1 < nch)
        def _():
            start_gather(c1 + 1, 0)
        wait_gather(1)
        start_write(c1, 1)
        return loss_math(c1, 1, acc)

    acc = lax.fori_loop(0, nch // 2, pair, jnp.zeros((16,), jnp.float32))
    wait_write(nch - 1, 1)
    acc_v[...] = acc
    pltpu.sync_copy(acc_v, partial.at[wid])


def _finish_body(in_ref, out_ref):
    out_ref[...] = in_ref[:, pl.ds(0, D)].reshape(RB // T_LEN, T_LEN, D)


def kernel(input_idx, target, embedding_table):
    B, T = input_idx.shape
    N = B * T
    fidx = input_idx.reshape(N)
    ftgt = target.reshape(N)
    table_p = jnp.pad(embedding_table, ((0, 0), (0, DP - D)))
    lse = pl.pallas_call(
        _lse_body,
        out_shape=jax.ShapeDtypeStruct((V, 1), jnp.float32),
    )(embedding_table).reshape(V)
    lse_p = jnp.pad(lse, (0, DP - V))

    mesh = plsc.VectorSubcoreMesh(core_axis_name="c", subcore_axis_name="s")
    sc = pl.kernel(
        _sc_body,
        out_type=[jax.ShapeDtypeStruct((N, DP), jnp.float32),
                  jax.ShapeDtypeStruct((NW, 16), jnp.float32)],
        mesh=mesh,
        scratch_types=[
            pltpu.VMEM((N // NW,), jnp.int32),
            pltpu.VMEM((N // NW,), jnp.int32),
            pltpu.VMEM((DP,), jnp.float32),
            pltpu.VMEM((CH, DP), jnp.float32),
            pltpu.VMEM((CH, DP), jnp.float32),
            pltpu.VMEM((16,), jnp.float32),
            pltpu.SemaphoreType.DMA,
            pltpu.SemaphoreType.DMA,
            pltpu.SemaphoreType.DMA,
            pltpu.SemaphoreType.DMA,
        ],
    )
    padded, partial = sc(table_p, lse_p, fidx, ftgt)

    nb = N // RB
    logits = pl.pallas_call(
        _finish_body,
        grid=(nb,),
        in_specs=[pl.BlockSpec((RB, DP), lambda i: (i, 0))],
        out_specs=pl.BlockSpec((RB // T_LEN, T_LEN, D), lambda i: (i, 0, 0)),
        out_shape=jax.ShapeDtypeStruct((B, T, D), jnp.float32),
    )(padded)
    loss = jnp.sum(partial) / jnp.float32(N)
    return logits, loss
